# baseline (speedup 1.0000x reference)
import functools

import jax
import jax.numpy as jnp
from jax import lax
from jax.experimental import pallas as pl
from jax.experimental.pallas import tpu as pltpu
from jax.experimental.pallas import tpu_sc as plsc

VOCAB = 1000
HIDDEN = 128
BATCH = 1024
SEQ = 50

NW = 32
ROWS_W = (BATCH * SEQ) // NW
C = 40
NCHUNK = ROWS_W // C


def _table_body(emb_ref, w_ref, b_ref, out_ref):
    out_ref[...] = (
        jnp.dot(emb_ref[...], w_ref[...], preferred_element_type=jnp.float32)
        + b_ref[...]
    )


def _make_table(embedding, W, b):
    return pl.pallas_call(
        _table_body,
        out_shape=jax.ShapeDtypeStruct((VOCAB, VOCAB), jnp.float32),
    )(embedding, W, b.reshape(1, VOCAB))


_sc_mesh = plsc.VectorSubcoreMesh(core_axis_name="c", subcore_axis_name="s")


@functools.partial(
    pl.kernel,
    mesh=_sc_mesh,
    out_type=jax.ShapeDtypeStruct((BATCH * SEQ, VOCAB), jnp.float32),
    compiler_params=pltpu.CompilerParams(use_tc_tiling_on_sc=False),
    scratch_types=[
        pltpu.VMEM((NCHUNK, C), jnp.int32),
        pltpu.VMEM((2, C, VOCAB), jnp.float32),
        pltpu.SemaphoreType.DMA,
        pltpu.SemaphoreType.DMA,
        pltpu.SemaphoreType.DMA,
        pltpu.SemaphoreType.DMA,
    ],
)
def _sc_gather(table, ids, out, idx_v, rows_v, gsem0, gsem1, ssem0, ssem1):
    cid = lax.axis_index("c")
    sid = lax.axis_index("s")
    wid = sid * 2 + cid
    base = wid * ROWS_W

    pltpu.sync_copy(ids.at[wid], idx_v)

    def gather(c, slot, sem):
        return pltpu.async_copy(table.at[idx_v.at[c]], rows_v.at[slot], sem)

    def gather_wait(c, slot, sem):
        pltpu.make_async_copy(table.at[idx_v.at[c]], rows_v.at[slot], sem).wait()

    def scatter(c, slot, sem):
        return pltpu.async_copy(
            rows_v.at[slot], out.at[pl.ds(base + c * C, C)], sem
        )

    def scatter_wait(c, slot, sem):
        pltpu.make_async_copy(
            rows_v.at[slot], out.at[pl.ds(base + c * C, C)], sem
        ).wait()

    gather(0, 0, gsem0)
    gather(1, 1, gsem1)

    def body(g, carry):
        c0 = 2 * g
        c1 = c0 + 1
        gather_wait(c0, 0, gsem0)
        scatter(c0, 0, ssem0)
        gather_wait(c1, 1, gsem1)
        scatter(c1, 1, ssem1)
        scatter_wait(c0, 0, ssem0)
        gather(c0 + 2, 0, gsem0)
        scatter_wait(c1, 1, ssem1)
        gather(c1 + 2, 1, gsem1)
        return carry

    lax.fori_loop(0, NCHUNK // 2 - 1, body, 0, unroll=False)

    c0 = NCHUNK - 2
    c1 = NCHUNK - 1
    gather_wait(c0, 0, gsem0)
    scatter(c0, 0, ssem0)
    gather_wait(c1, 1, gsem1)
    scatter(c1, 1, ssem1)
    scatter_wait(c0, 0, ssem0)
    scatter_wait(c1, 1, ssem1)


def kernel(input_ids, embedding, W, b):
    table = _make_table(embedding, W, b)
    ids = input_ids.astype(jnp.int32).reshape(NW, NCHUNK, C)
    flat = _sc_gather(table, ids)
    return flat.reshape(BATCH, SEQ, VOCAB)


# speedup vs baseline: 1.6911x; 1.6911x over previous
import functools

import jax
import jax.numpy as jnp
from jax import lax
from jax.experimental import pallas as pl
from jax.experimental.pallas import tpu as pltpu
from jax.experimental.pallas import tpu_sc as plsc

VOCAB = 1000
VPAD = 1024
NPIECE = VPAD // 128
VMAIN = 896
VTAIL = VOCAB - VMAIN
HIDDEN = 128
BATCH = 1024
SEQ = 50

NW = 32
BATCH_W = BATCH // NW


def _table_body(emb_ref, w_ref, b_ref, out_ref):
    out_ref[0] = (
        jnp.dot(emb_ref[...], w_ref[...], preferred_element_type=jnp.float32)
        + b_ref[...]
    )


def _make_table(embedding, W, b):
    w_pad = jnp.zeros((HIDDEN, VPAD), jnp.float32).at[:, :VOCAB].set(W)
    b_pad = jnp.zeros((1, VPAD), jnp.float32).at[:, :VOCAB].set(b.reshape(1, VOCAB))
    return pl.pallas_call(
        _table_body,
        grid=(NPIECE,),
        in_specs=[
            pl.BlockSpec((VOCAB, HIDDEN), lambda p: (0, 0)),
            pl.BlockSpec((HIDDEN, 128), lambda p: (0, p)),
            pl.BlockSpec((1, 128), lambda p: (0, p)),
        ],
        out_specs=pl.BlockSpec((1, VOCAB, 128), lambda p: (p, 0, 0)),
        out_shape=jax.ShapeDtypeStruct((NPIECE, VOCAB, 128), jnp.float32),
    )(embedding, w_pad, b_pad)


_sc_mesh = plsc.VectorSubcoreMesh(core_axis_name="c", subcore_axis_name="s")


@functools.partial(
    pl.kernel,
    mesh=_sc_mesh,
    out_type=jax.ShapeDtypeStruct((BATCH, SEQ, VOCAB), jnp.float32),
    scratch_types=[
        pltpu.VMEM((SEQ,), jnp.int32),
        pltpu.VMEM((SEQ,), jnp.int32),
        pltpu.VMEM((2, NPIECE, SEQ, 128), jnp.float32),
        pltpu.VMEM((2, SEQ, VTAIL), jnp.float32),
        pltpu.SemaphoreType.DMA,
        pltpu.SemaphoreType.DMA,
        pltpu.SemaphoreType.DMA,
        pltpu.SemaphoreType.DMA,
        pltpu.SemaphoreType.DMA,
        pltpu.SemaphoreType.DMA,
    ],
)
def _sc_gather(
    table, ids, out, idxA, idxB, rows_v, tail_v, g0, g1, s0, s1, i0, i1
):
    cid = lax.axis_index("c")
    sid = lax.axis_index("s")
    wid = sid * 2 + cid
    base = wid * BATCH_W

    def idx_fetch(c, idx, sem):
        pltpu.async_copy(ids.at[wid, c], idx, sem)

    def idx_wait(c, idx, sem):
        pltpu.make_async_copy(ids.at[wid, c], idx, sem).wait()

    def gather(idx, slot, sem):
        for p in range(NPIECE):
            pltpu.async_copy(table.at[p].at[idx], rows_v.at[slot, p], sem)

    def gather_wait(idx, slot, sem):
        for p in range(NPIECE):
            pltpu.make_async_copy(
                table.at[p].at[idx], rows_v.at[slot, p], sem
            ).wait()

    def compact_tail(slot):
        def row(r, carry):
            for k in range(6):
                tail_v[slot, r, pl.ds(16 * k, 16)] = rows_v[
                    slot, NPIECE - 1, r, pl.ds(16 * k, 16)
                ]
            tail_v[slot, r, pl.ds(VTAIL - 16, 16)] = rows_v[
                slot, NPIECE - 1, r, pl.ds(VTAIL - 16, 16)
            ]
            return carry

        lax.fori_loop(0, SEQ, row, 0, unroll=False)

    def scatter(c, slot, sem):
        for p in range(NPIECE - 1):
            pltpu.async_copy(
                rows_v.at[slot, p],
                out.at[base + c, :, pl.ds(128 * p, 128)],
                sem,
            )
        pltpu.async_copy(
            tail_v.at[slot], out.at[base + c, :, pl.ds(VMAIN, VTAIL)], sem
        )

    def scatter_wait(c, slot, sem):
        for p in range(NPIECE - 1):
            pltpu.make_async_copy(
                rows_v.at[slot, p],
                out.at[base + c, :, pl.ds(128 * p, 128)],
                sem,
            ).wait()
        pltpu.make_async_copy(
            tail_v.at[slot], out.at[base + c, :, pl.ds(VMAIN, VTAIL)], sem
        ).wait()

    pltpu.sync_copy(ids.at[wid, 0], idxA)
    pltpu.sync_copy(ids.at[wid, 1], idxB)
    gather(idxA, 0, g0)
    gather(idxB, 1, g1)

    def body(g, carry):
        c0 = 2 * g
        c1 = c0 + 1
        gather_wait(idxA, 0, g0)
        idx_fetch(c0 + 2, idxA, i0)
        compact_tail(0)
        scatter(c0, 0, s0)
        gather_wait(idxB, 1, g1)
        idx_fetch(c1 + 2, idxB, i1)
        compact_tail(1)
        scatter(c1, 1, s1)
        scatter_wait(c0, 0, s0)
        idx_wait(c0 + 2, idxA, i0)
        gather(idxA, 0, g0)
        scatter_wait(c1, 1, s1)
        idx_wait(c1 + 2, idxB, i1)
        gather(idxB, 1, g1)
        return carry

    lax.fori_loop(0, BATCH_W // 2 - 1, body, 0, unroll=False)

    c0 = BATCH_W - 2
    c1 = BATCH_W - 1
    gather_wait(idxA, 0, g0)
    compact_tail(0)
    scatter(c0, 0, s0)
    gather_wait(idxB, 1, g1)
    compact_tail(1)
    scatter(c1, 1, s1)
    scatter_wait(c0, 0, s0)
    scatter_wait(c1, 1, s1)


def kernel(input_ids, embedding, W, b):
    table = _make_table(embedding, W, b)
    ids = input_ids.astype(jnp.int32).reshape(NW, BATCH_W, SEQ)
    return _sc_gather(table, ids)


# speedup vs baseline: 4.9783x; 2.9438x over previous
import jax
import jax.numpy as jnp
from jax import lax
from jax.experimental import pallas as pl
from jax.experimental.pallas import tpu as pltpu
from jax.experimental.pallas import tpu_sc as plsc
import functools

VOCAB = 1000
HIDDEN = 128
BATCH = 1024
SEQ = 50

NW = 32
BATCH_W = BATCH // NW


_sc_mesh = plsc.VectorSubcoreMesh(core_axis_name="c", subcore_axis_name="s")


@functools.partial(
    pl.kernel,
    mesh=_sc_mesh,
    out_type=jax.ShapeDtypeStruct((BATCH, SEQ, HIDDEN), jnp.float32),
    scratch_types=[
        pltpu.VMEM((SEQ,), jnp.int32),
        pltpu.VMEM((SEQ,), jnp.int32),
        pltpu.VMEM((2, SEQ, HIDDEN), jnp.float32),
        pltpu.SemaphoreType.DMA,
        pltpu.SemaphoreType.DMA,
        pltpu.SemaphoreType.DMA,
        pltpu.SemaphoreType.DMA,
        pltpu.SemaphoreType.DMA,
        pltpu.SemaphoreType.DMA,
    ],
)
def _sc_gather(emb, ids, out, idxA, idxB, rows_v, g0, g1, s0, s1, i0, i1):
    cid = lax.axis_index("c")
    sid = lax.axis_index("s")
    wid = sid * 2 + cid
    base = wid * BATCH_W

    def idx_fetch(c, idx, sem):
        pltpu.async_copy(ids.at[base + c], idx, sem)

    def idx_wait(c, idx, sem):
        pltpu.make_async_copy(ids.at[base + c], idx, sem).wait()

    def gather(idx, slot, sem):
        pltpu.async_copy(emb.at[idx], rows_v.at[slot], sem)

    def gather_wait(idx, slot, sem):
        pltpu.make_async_copy(emb.at[idx], rows_v.at[slot], sem).wait()

    def scatter(c, slot, sem):
        pltpu.async_copy(rows_v.at[slot], out.at[base + c], sem)

    def scatter_wait(c, slot, sem):
        pltpu.make_async_copy(rows_v.at[slot], out.at[base + c], sem).wait()

    pltpu.sync_copy(ids.at[base], idxA)
    pltpu.sync_copy(ids.at[base + 1], idxB)
    gather(idxA, 0, g0)
    gather(idxB, 1, g1)

    def body(g, carry):
        c0 = 2 * g
        c1 = c0 + 1
        gather_wait(idxA, 0, g0)
        idx_fetch(c0 + 2, idxA, i0)
        scatter(c0, 0, s0)
        gather_wait(idxB, 1, g1)
        idx_fetch(c1 + 2, idxB, i1)
        scatter(c1, 1, s1)
        scatter_wait(c0, 0, s0)
        idx_wait(c0 + 2, idxA, i0)
        gather(idxA, 0, g0)
        scatter_wait(c1, 1, s1)
        idx_wait(c1 + 2, idxB, i1)
        gather(idxB, 1, g1)
        return carry

    lax.fori_loop(0, BATCH_W // 2 - 1, body, 0, unroll=False)

    gather_wait(idxA, 0, g0)
    scatter(BATCH_W - 2, 0, s0)
    gather_wait(idxB, 1, g1)
    scatter(BATCH_W - 1, 1, s1)
    scatter_wait(BATCH_W - 2, 0, s0)
    scatter_wait(BATCH_W - 1, 1, s1)


def _mm_body(wt_ref, b_ref, emb_ref, out_ref):
    s = pl.program_id(0)
    e = emb_ref[:, s, :]
    out_ref[0] = (
        jax.lax.dot_general(
            wt_ref[...], e, (((1,), (1,)), ((), ())),
            preferred_element_type=jnp.float32,
        )
        + b_ref[...]
    )


def _matmul(WT, b_col, emb_g):
    return pl.pallas_call(
        _mm_body,
        grid=(SEQ,),
        compiler_params=pltpu.CompilerParams(
            vmem_limit_bytes=48 * 1024 * 1024
        ),
        in_specs=[
            pl.BlockSpec((VOCAB, HIDDEN), lambda s: (0, 0)),
            pl.BlockSpec((VOCAB, 1), lambda s: (0, 0)),
            pl.BlockSpec((BATCH, SEQ, HIDDEN), lambda s: (0, 0, 0)),
        ],
        out_specs=pl.BlockSpec((1, VOCAB, BATCH), lambda s: (s, 0, 0)),
        out_shape=jax.ShapeDtypeStruct((SEQ, VOCAB, BATCH), jnp.float32),
    )(WT, b_col, emb_g)


def kernel(input_ids, embedding, W, b):
    ids = input_ids.astype(jnp.int32)
    emb_g = _sc_gather(embedding, ids)
    out_t = _matmul(W.T, b.reshape(VOCAB, 1), emb_g)
    return out_t.transpose(2, 0, 1)
